# Initial kernel scaffold; baseline (speedup 1.0000x reference)
import jax
import jax.numpy as jnp
from jax.experimental import pallas as pl


def kernel(points, gt_bboxes, gt_labels, regress_ranges, strides_per_point):
    raise NotImplementedError("write your pallas kernel here")


# speedup vs baseline: 4.2483x; 4.2483x over previous
import functools

import jax
import jax.numpy as jnp
from jax import lax
from jax.experimental import pallas as pl
from jax.experimental.pallas import tpu as pltpu
from jax.experimental.pallas import tpu_sc as plsc

_INF = 100000000.0
_BACKGROUND = 8
_RADIUS = 1.5

_N = 21824
_NW = 32
_PER_W = 688
_NP = _NW * _PER_W
_NVEC = _PER_W // 16
_G = 100
_GP = 112


def _sqrt16(x):
    i = plsc.bitcast(x, jnp.int32)
    y = plsc.bitcast(jnp.int32(0x5F3759DF) - (i >> 1), jnp.float32)
    for _ in range(3):
        y = y * (1.5 - 0.5 * x * y * y)
    return x * y


def _body(xs_h, ys_h, st_h, rlo_h, rhi_h, gx1_h, gy1_h, gx2_h, gy2_h, glab_h,
          lab_o, bl_o, bt_o, br_o, bb_o, ctr_o,
          xs_v, ys_v, st_v, rlo_v, rhi_v,
          gx1_v, gy1_v, gx2_v, gy2_v, cx_v, cy_v, ar_v, glab_v,
          lab_v, bl_v, bt_v, br_v, bb_v, ctr_v):
    wid = lax.axis_index("s") * 2 + lax.axis_index("c")
    base = wid * _PER_W

    pltpu.sync_copy(xs_h.at[pl.ds(base, _PER_W)], xs_v)
    pltpu.sync_copy(ys_h.at[pl.ds(base, _PER_W)], ys_v)
    pltpu.sync_copy(st_h.at[pl.ds(base, _PER_W)], st_v)
    pltpu.sync_copy(rlo_h.at[pl.ds(base, _PER_W)], rlo_v)
    pltpu.sync_copy(rhi_h.at[pl.ds(base, _PER_W)], rhi_v)
    pltpu.sync_copy(gx1_h, gx1_v)
    pltpu.sync_copy(gy1_h, gy1_v)
    pltpu.sync_copy(gx2_h, gx2_v)
    pltpu.sync_copy(gy2_h, gy2_v)
    pltpu.sync_copy(glab_h, glab_v)

    for j in range(_GP // 16):
        sl = pl.ds(j * 16, 16)
        x1 = gx1_v[sl]
        y1 = gy1_v[sl]
        x2 = gx2_v[sl]
        y2 = gy2_v[sl]
        cx_v[sl] = (x1 + x2) * 0.5
        cy_v[sl] = (y1 + y2) * 0.5
        ar_v[sl] = (x2 - x1) * (y2 - y1)

    def point_vec(i, _):
        off = i * 16
        sl = pl.ds(off, 16)
        xs = xs_v[sl]
        ys = ys_v[sl]
        rad = st_v[sl] * _RADIUS
        rlo = rlo_v[sl]
        rhi = rhi_v[sl]

        def per_gt(g, carry):
            min_area, min_idx = carry
            gi = jnp.full((16,), g, jnp.int32)
            x1 = plsc.load_gather(gx1_v, [gi])
            y1 = plsc.load_gather(gy1_v, [gi])
            x2 = plsc.load_gather(gx2_v, [gi])
            y2 = plsc.load_gather(gy2_v, [gi])
            cx = plsc.load_gather(cx_v, [gi])
            cy = plsc.load_gather(cy_v, [gi])
            ar = plsc.load_gather(ar_v, [gi])
            l = xs - x1
            t = ys - y1
            r = x2 - xs
            b = y2 - ys
            maxreg = jnp.maximum(jnp.maximum(l, t), jnp.maximum(r, b))
            in_rr = (maxreg >= rlo) & (maxreg <= rhi)
            cgx1 = jnp.maximum(cx - rad, x1)
            cgy1 = jnp.maximum(cy - rad, y1)
            cgx2 = jnp.minimum(cx + rad, x2)
            cgy2 = jnp.minimum(cy + rad, y2)
            m = jnp.minimum(jnp.minimum(xs - cgx1, ys - cgy1),
                            jnp.minimum(cgx2 - xs, cgy2 - ys))
            cond = (m > 0) & in_rr
            a_m = jnp.where(cond, ar, _INF)
            better = a_m < min_area
            return jnp.minimum(min_area, a_m), jnp.where(better, gi, min_idx)

        init = (jnp.full((16,), _INF, jnp.float32), jnp.zeros((16,), jnp.int32))
        min_area, min_idx = lax.fori_loop(0, _G, per_gt, init)

        wx1 = plsc.load_gather(gx1_v, [min_idx])
        wy1 = plsc.load_gather(gy1_v, [min_idx])
        wx2 = plsc.load_gather(gx2_v, [min_idx])
        wy2 = plsc.load_gather(gy2_v, [min_idx])
        wl = plsc.load_gather(glab_v, [min_idx])
        l = xs - wx1
        t = ys - wy1
        r = wx2 - xs
        b = wy2 - ys
        lr_min = jnp.minimum(l, r)
        lr_max = jnp.maximum(l, r)
        tb_min = jnp.minimum(t, b)
        tb_max = jnp.maximum(t, b)
        ratio = (lr_min / jnp.maximum(lr_max, 1e-6)) * (tb_min / jnp.maximum(tb_max, 1e-6))
        ctr = _sqrt16(jnp.maximum(ratio, 1e-12))
        is_bg = min_area >= _INF
        lab = jnp.where(is_bg, _BACKGROUND, wl)
        s = st_v[sl]
        lab_v[sl] = lab
        bl_v[sl] = l / s
        bt_v[sl] = t / s
        br_v[sl] = r / s
        bb_v[sl] = b / s
        ctr_v[sl] = ctr
        return _

    lax.fori_loop(0, _NVEC, point_vec, 0)

    pltpu.sync_copy(lab_v, lab_o.at[pl.ds(base, _PER_W)])
    pltpu.sync_copy(bl_v, bl_o.at[pl.ds(base, _PER_W)])
    pltpu.sync_copy(bt_v, bt_o.at[pl.ds(base, _PER_W)])
    pltpu.sync_copy(br_v, br_o.at[pl.ds(base, _PER_W)])
    pltpu.sync_copy(bb_v, bb_o.at[pl.ds(base, _PER_W)])
    pltpu.sync_copy(ctr_v, ctr_o.at[pl.ds(base, _PER_W)])


_f32 = jnp.float32
_i32 = jnp.int32

_sc_call = pl.kernel(
    _body,
    out_type=tuple(
        jax.ShapeDtypeStruct((_NP,), dt)
        for dt in (_i32, _f32, _f32, _f32, _f32, _f32)
    ),
    mesh=plsc.VectorSubcoreMesh(
        core_axis_name="c", subcore_axis_name="s", num_cores=2, num_subcores=16
    ),
    compiler_params=pltpu.CompilerParams(needs_layout_passes=False),
    scratch_types=[
        pltpu.VMEM((_PER_W,), _f32),
        pltpu.VMEM((_PER_W,), _f32),
        pltpu.VMEM((_PER_W,), _f32),
        pltpu.VMEM((_PER_W,), _f32),
        pltpu.VMEM((_PER_W,), _f32),
        pltpu.VMEM((_GP,), _f32),
        pltpu.VMEM((_GP,), _f32),
        pltpu.VMEM((_GP,), _f32),
        pltpu.VMEM((_GP,), _f32),
        pltpu.VMEM((_GP,), _f32),
        pltpu.VMEM((_GP,), _f32),
        pltpu.VMEM((_GP,), _f32),
        pltpu.VMEM((_GP,), _i32),
        pltpu.VMEM((_PER_W,), _i32),
        pltpu.VMEM((_PER_W,), _f32),
        pltpu.VMEM((_PER_W,), _f32),
        pltpu.VMEM((_PER_W,), _f32),
        pltpu.VMEM((_PER_W,), _f32),
        pltpu.VMEM((_PER_W,), _f32),
    ],
)


def kernel(points, gt_bboxes, gt_labels, regress_ranges, strides_per_point):
    padn = _NP - _N
    xs = jnp.pad(points[:, 0], (0, padn), constant_values=-1e9)
    ys = jnp.pad(points[:, 1], (0, padn), constant_values=-1e9)
    st = jnp.pad(strides_per_point, (0, padn), constant_values=1.0)
    rlo = jnp.pad(regress_ranges[:, 0], (0, padn), constant_values=0.0)
    rhi = jnp.pad(regress_ranges[:, 1], (0, padn), constant_values=-1.0)
    padg = _GP - _G
    gx1 = jnp.pad(gt_bboxes[:, 0], (0, padg))
    gy1 = jnp.pad(gt_bboxes[:, 1], (0, padg))
    gx2 = jnp.pad(gt_bboxes[:, 2], (0, padg))
    gy2 = jnp.pad(gt_bboxes[:, 3], (0, padg))
    glab = jnp.pad(gt_labels.astype(_i32), (0, padg))

    lab, bl, bt, br, bb, ctr = _sc_call(
        xs, ys, st, rlo, rhi, gx1, gy1, gx2, gy2, glab)

    bbox = jnp.stack([bl[:_N], bt[:_N], br[:_N], bb[:_N]], axis=-1)
    return lab[:_N], bbox, ctr[:_N]


# speedup vs baseline: 6.2231x; 1.4648x over previous
import functools

import jax
import jax.numpy as jnp
from jax import lax
from jax.experimental import pallas as pl
from jax.experimental.pallas import tpu as pltpu
from jax.experimental.pallas import tpu_sc as plsc

_INF = 100000000.0
_BACKGROUND = 8
_RADIUS = 1.5

_N = 21824
_NW = 32
_PER_W = 688
_NP = _NW * _PER_W
_NVEC = _PER_W // 16
_G = 100
_GP = 112


def _sqrt16(x):
    i = plsc.bitcast(x, jnp.int32)
    y = plsc.bitcast(jnp.int32(0x5F3759DF) - (i >> 1), jnp.float32)
    for _ in range(3):
        y = y * (1.5 - 0.5 * x * y * y)
    return x * y


def _body(xs_h, ys_h, st_h, rlo_h, rhi_h, gx1_h, gy1_h, gx2_h, gy2_h, glab_h,
          lab_o, bl_o, bt_o, br_o, bb_o, ctr_o,
          xs_v, ys_v, st_v, rlo_v, rhi_v,
          gx1_v, gy1_v, gx2_v, gy2_v, cx_v, cy_v, ar_v, hm_v, glab_v, gidx_v,
          lab_v, bl_v, bt_v, br_v, bb_v, ctr_v):
    wid = lax.axis_index("s") * 2 + lax.axis_index("c")
    base = wid * _PER_W

    pltpu.sync_copy(xs_h.at[pl.ds(base, _PER_W)], xs_v)
    pltpu.sync_copy(ys_h.at[pl.ds(base, _PER_W)], ys_v)
    pltpu.sync_copy(st_h.at[pl.ds(base, _PER_W)], st_v)
    pltpu.sync_copy(rlo_h.at[pl.ds(base, _PER_W)], rlo_v)
    pltpu.sync_copy(rhi_h.at[pl.ds(base, _PER_W)], rhi_v)
    pltpu.sync_copy(gx1_h, gx1_v)
    pltpu.sync_copy(gy1_h, gy1_v)
    pltpu.sync_copy(gx2_h, gx2_v)
    pltpu.sync_copy(gy2_h, gy2_v)
    pltpu.sync_copy(glab_h, glab_v)

    for j in range(_GP // 16):
        sl = pl.ds(j * 16, 16)
        x1 = gx1_v[sl]
        y1 = gy1_v[sl]
        x2 = gx2_v[sl]
        y2 = gy2_v[sl]
        cx_v[sl] = (x1 + x2) * 0.5
        cy_v[sl] = (y1 + y2) * 0.5
        ar_v[sl] = (x2 - x1) * (y2 - y1)
        hm_v[sl] = jnp.maximum(x2 - x1, y2 - y1)

    def point_vec(i, _):
        off = i * 16
        sl = pl.ds(off, 16)
        xs = xs_v[sl]
        ys = ys_v[sl]
        rad = st_v[sl] * _RADIUS
        rlo = rlo_v[sl]
        rhi = rhi_v[sl]

        pxmn = jnp.min(xs)
        pxmx = jnp.max(xs)
        pymn = jnp.min(ys)
        pymx = jnp.max(ys)
        radv = jnp.max(rad)
        rlomn = jnp.min(rlo)
        rhimx2 = jnp.max(rhi) * 2.0
        tot = jnp.int32(0)
        for j in range(_GP // 16):
            gsl = pl.ds(j * 16, 16)
            x1 = gx1_v[gsl]
            y1 = gy1_v[gsl]
            x2 = gx2_v[gsl]
            y2 = gy2_v[gsl]
            cx = cx_v[gsl]
            cy = cy_v[gsl]
            hm = hm_v[gsl]
            cgx1 = jnp.maximum(cx - radv, x1)
            cgx2 = jnp.minimum(cx + radv, x2)
            cgy1 = jnp.maximum(cy - radv, y1)
            cgy2 = jnp.minimum(cy + radv, y2)
            keep = (cgx1 < pxmx) & (cgx2 > pxmn) & (cgy1 < pymx) & (cgy2 > pymn)
            keep &= (hm >= rlomn) & (hm <= rhimx2)
            gvec = jnp.arange(16, dtype=jnp.int32) + (j * 16)
            keep &= gvec < _G
            plsc.store_compressed(gidx_v.at[pl.ds(tot, 16)], gvec, mask=keep)
            tot = tot + jnp.sum(keep.astype(jnp.int32))

        def per_gt(k, carry):
            min_area, min_idx = carry
            ki = jnp.full((16,), k, jnp.int32)
            gi = plsc.load_gather(gidx_v, [ki])
            x1 = plsc.load_gather(gx1_v, [gi])
            y1 = plsc.load_gather(gy1_v, [gi])
            x2 = plsc.load_gather(gx2_v, [gi])
            y2 = plsc.load_gather(gy2_v, [gi])
            cx = plsc.load_gather(cx_v, [gi])
            cy = plsc.load_gather(cy_v, [gi])
            ar = plsc.load_gather(ar_v, [gi])
            l = xs - x1
            t = ys - y1
            r = x2 - xs
            b = y2 - ys
            maxreg = jnp.maximum(jnp.maximum(l, t), jnp.maximum(r, b))
            in_rr = (maxreg >= rlo) & (maxreg <= rhi)
            cgx1 = jnp.maximum(cx - rad, x1)
            cgy1 = jnp.maximum(cy - rad, y1)
            cgx2 = jnp.minimum(cx + rad, x2)
            cgy2 = jnp.minimum(cy + rad, y2)
            m = jnp.minimum(jnp.minimum(xs - cgx1, ys - cgy1),
                            jnp.minimum(cgx2 - xs, cgy2 - ys))
            cond = (m > 0) & in_rr
            a_m = jnp.where(cond, ar, _INF)
            better = a_m < min_area
            return jnp.minimum(min_area, a_m), jnp.where(better, gi, min_idx)

        init = (jnp.full((16,), _INF, jnp.float32), jnp.zeros((16,), jnp.int32))
        min_area, min_idx = lax.fori_loop(0, tot, per_gt, init)

        wx1 = plsc.load_gather(gx1_v, [min_idx])
        wy1 = plsc.load_gather(gy1_v, [min_idx])
        wx2 = plsc.load_gather(gx2_v, [min_idx])
        wy2 = plsc.load_gather(gy2_v, [min_idx])
        wl = plsc.load_gather(glab_v, [min_idx])
        l = xs - wx1
        t = ys - wy1
        r = wx2 - xs
        b = wy2 - ys
        lr_min = jnp.minimum(l, r)
        lr_max = jnp.maximum(l, r)
        tb_min = jnp.minimum(t, b)
        tb_max = jnp.maximum(t, b)
        ratio = (lr_min / jnp.maximum(lr_max, 1e-6)) * (tb_min / jnp.maximum(tb_max, 1e-6))
        ctr = _sqrt16(jnp.maximum(ratio, 1e-12))
        is_bg = min_area >= _INF
        lab = jnp.where(is_bg, _BACKGROUND, wl)
        s = st_v[sl]
        lab_v[sl] = lab
        bl_v[sl] = l / s
        bt_v[sl] = t / s
        br_v[sl] = r / s
        bb_v[sl] = b / s
        ctr_v[sl] = ctr
        return _

    lax.fori_loop(0, _NVEC, point_vec, 0)

    pltpu.sync_copy(lab_v, lab_o.at[pl.ds(base, _PER_W)])
    pltpu.sync_copy(bl_v, bl_o.at[pl.ds(base, _PER_W)])
    pltpu.sync_copy(bt_v, bt_o.at[pl.ds(base, _PER_W)])
    pltpu.sync_copy(br_v, br_o.at[pl.ds(base, _PER_W)])
    pltpu.sync_copy(bb_v, bb_o.at[pl.ds(base, _PER_W)])
    pltpu.sync_copy(ctr_v, ctr_o.at[pl.ds(base, _PER_W)])


_f32 = jnp.float32
_i32 = jnp.int32

_sc_call = pl.kernel(
    _body,
    out_type=tuple(
        jax.ShapeDtypeStruct((_NP,), dt)
        for dt in (_i32, _f32, _f32, _f32, _f32, _f32)
    ),
    mesh=plsc.VectorSubcoreMesh(
        core_axis_name="c", subcore_axis_name="s", num_cores=2, num_subcores=16
    ),
    compiler_params=pltpu.CompilerParams(needs_layout_passes=False),
    scratch_types=[
        pltpu.VMEM((_PER_W,), _f32),
        pltpu.VMEM((_PER_W,), _f32),
        pltpu.VMEM((_PER_W,), _f32),
        pltpu.VMEM((_PER_W,), _f32),
        pltpu.VMEM((_PER_W,), _f32),
        pltpu.VMEM((_GP,), _f32),
        pltpu.VMEM((_GP,), _f32),
        pltpu.VMEM((_GP,), _f32),
        pltpu.VMEM((_GP,), _f32),
        pltpu.VMEM((_GP,), _f32),
        pltpu.VMEM((_GP,), _f32),
        pltpu.VMEM((_GP,), _f32),
        pltpu.VMEM((_GP,), _f32),
        pltpu.VMEM((_GP,), _i32),
        pltpu.VMEM((128,), _i32),
        pltpu.VMEM((_PER_W,), _i32),
        pltpu.VMEM((_PER_W,), _f32),
        pltpu.VMEM((_PER_W,), _f32),
        pltpu.VMEM((_PER_W,), _f32),
        pltpu.VMEM((_PER_W,), _f32),
        pltpu.VMEM((_PER_W,), _f32),
    ],
)


def kernel(points, gt_bboxes, gt_labels, regress_ranges, strides_per_point):
    padn = _NP - _N
    xs = jnp.pad(points[:, 0], (0, padn), constant_values=-1e9)
    ys = jnp.pad(points[:, 1], (0, padn), constant_values=-1e9)
    st = jnp.pad(strides_per_point, (0, padn), constant_values=1.0)
    rlo = jnp.pad(regress_ranges[:, 0], (0, padn), constant_values=0.0)
    rhi = jnp.pad(regress_ranges[:, 1], (0, padn), constant_values=-1.0)
    padg = _GP - _G
    gx1 = jnp.pad(gt_bboxes[:, 0], (0, padg))
    gy1 = jnp.pad(gt_bboxes[:, 1], (0, padg))
    gx2 = jnp.pad(gt_bboxes[:, 2], (0, padg))
    gy2 = jnp.pad(gt_bboxes[:, 3], (0, padg))
    glab = jnp.pad(gt_labels.astype(_i32), (0, padg))

    lab, bl, bt, br, bb, ctr = _sc_call(
        xs, ys, st, rlo, rhi, gx1, gy1, gx2, gy2, glab)

    bbox = jnp.stack([bl[:_N], bt[:_N], br[:_N], bb[:_N]], axis=-1)
    return lab[:_N], bbox, ctr[:_N]
